# Initial kernel scaffold; baseline (speedup 1.0000x reference)
import jax
import jax.numpy as jnp
from jax.experimental import pallas as pl


def kernel(x, edge_index, batch, W1, b1, W2, b2, Wc, bc):
    raise NotImplementedError("write your pallas kernel here")


# speedup vs baseline: 10.4432x; 10.4432x over previous
import functools

import jax
import jax.numpy as jnp
from jax import lax
from jax.experimental import pallas as pl
from jax.experimental.pallas import tpu as pltpu
from jax.experimental.pallas import tpu_sc as plsc

_NC = 2
_NS = 16
_NW = _NC * _NS
_L = 16
_B = 128
_BLK = 2048
_G = 64


def _sc_mesh():
    return plsc.VectorSubcoreMesh(
        core_axis_name="c", subcore_axis_name="s",
        num_cores=_NC, num_subcores=_NS)


def _make_deg(Npad, Epad):
    EW = Epad // _NW
    nchunks = EW // _B
    RPT = Npad // _NS

    @functools.partial(
        pl.kernel,
        out_type=jax.ShapeDtypeStruct((_NC, Npad), jnp.float32),
        mesh=_sc_mesh(),
        scratch_types=[
            pltpu.VMEM((_B,), jnp.int32),
            pltpu.VMEM((_B,), jnp.float32),
            pltpu.VMEM((RPT,), jnp.float32),
            pltpu.VMEM_SHARED((Npad,), jnp.float32),
        ],
    )
    def deg(dst_hbm, out_hbm, idx_d, ones_v, zeros_v, acc):
        c = lax.axis_index("c")
        s = lax.axis_index("s")
        wid = s * _NC + c
        for i in range(_B // _L):
            ones_v[pl.ds(i * _L, _L)] = jnp.full((_L,), 1.0, jnp.float32)
        for i in range(RPT // _L):
            zeros_v[pl.ds(i * _L, _L)] = jnp.zeros((_L,), jnp.float32)
        pltpu.sync_copy(zeros_v, acc.at[pl.ds(s * RPT, RPT)])
        plsc.subcore_barrier()
        base0 = wid * EW

        def body(j, carry):
            pltpu.sync_copy(dst_hbm.at[pl.ds(base0 + j * _B, _B)], idx_d)
            pltpu.sync_copy(ones_v, acc.at[idx_d], add=True)
            return carry

        lax.fori_loop(0, nchunks, body, 0)
        plsc.subcore_barrier()
        pltpu.sync_copy(acc.at[pl.ds(s * RPT, RPT)],
                        out_hbm.at[c, pl.ds(s * RPT, RPT)])

    return deg


def _make_agg(Npad, D, Epad):
    EW = Epad // _NW
    nchunks = EW // _B
    RPT = Npad // _NS

    @functools.partial(
        pl.kernel,
        out_type=jax.ShapeDtypeStruct((_NC, Npad, D), jnp.float32),
        mesh=_sc_mesh(),
        scratch_types=[
            pltpu.VMEM((_B,), jnp.int32),
            pltpu.VMEM((_B,), jnp.int32),
            pltpu.VMEM((_B, D), jnp.float32),
            pltpu.VMEM_SHARED((Npad, D), jnp.float32),
            pltpu.SemaphoreType.DMA,
        ],
    )
    def agg(p_hbm, src_hbm, dst_hbm, out_hbm, idx_s, idx_d, rows, acc, sem):
        c = lax.axis_index("c")
        s = lax.axis_index("s")
        wid = s * _NC + c
        pltpu.sync_copy(p_hbm.at[pl.ds(s * RPT, RPT)],
                        acc.at[pl.ds(s * RPT, RPT)])
        plsc.subcore_barrier()
        base0 = wid * EW

        def body(j, carry):
            base = base0 + j * _B
            pltpu.sync_copy(src_hbm.at[pl.ds(base, _B)], idx_s)
            pltpu.sync_copy(dst_hbm.at[pl.ds(base, _B)], idx_d)
            pltpu.async_copy(p_hbm.at[idx_s], rows, sem).wait()
            pltpu.sync_copy(rows, acc.at[idx_d], add=True)
            return carry

        lax.fori_loop(0, nchunks, body, 0)
        plsc.subcore_barrier()
        pltpu.sync_copy(acc.at[pl.ds(s * RPT, RPT)],
                        out_hbm.at[c, pl.ds(s * RPT, RPT)])

    return agg


def _mm_scale(x_p, W, degT, N):
    Npad, D = x_p.shape
    H = W.shape[1]
    grid = Npad // _BLK

    def body(x_ref, w_ref, deg_ref, out_ref):
        i = pl.program_id(0)
        d = jnp.sum(deg_ref[...], axis=1, keepdims=True) + 1.0
        rows = lax.broadcasted_iota(jnp.int32, (_BLK, 1), 0) + i * _BLK
        dinv = jnp.where(rows < N, lax.rsqrt(d), 0.0)
        out_ref[...] = jnp.dot(x_ref[...], w_ref[...],
                               preferred_element_type=jnp.float32) * dinv

    return pl.pallas_call(
        body,
        grid=(grid,),
        in_specs=[
            pl.BlockSpec((_BLK, D), lambda i: (i, 0)),
            pl.BlockSpec((D, H), lambda i: (0, 0)),
            pl.BlockSpec((_BLK, _NC), lambda i: (i, 0)),
        ],
        out_specs=pl.BlockSpec((_BLK, H), lambda i: (i, 0)),
        out_shape=jax.ShapeDtypeStruct((Npad, H), jnp.float32),
    )(x_p, W, degT)


def _layer_mm(aggp, p_prev, degT, b, W, N):
    Npad, H = p_prev.shape
    grid = Npad // _BLK

    def body(a_ref, p_ref, deg_ref, b_ref, w_ref, out_ref):
        i = pl.program_id(0)
        d = jnp.sum(deg_ref[...], axis=1, keepdims=True) + 1.0
        rows = lax.broadcasted_iota(jnp.int32, (_BLK, 1), 0) + i * _BLK
        dinv = jnp.where(rows < N, lax.rsqrt(d), 0.0)
        h = jnp.maximum(dinv * (a_ref[0] + a_ref[1] - p_ref[...]) + b_ref[...],
                        0.0)
        out_ref[...] = jnp.dot(h, w_ref[...],
                               preferred_element_type=jnp.float32) * dinv

    return pl.pallas_call(
        body,
        grid=(grid,),
        in_specs=[
            pl.BlockSpec((_NC, _BLK, H), lambda i: (0, i, 0)),
            pl.BlockSpec((_BLK, H), lambda i: (i, 0)),
            pl.BlockSpec((_BLK, _NC), lambda i: (i, 0)),
            pl.BlockSpec((1, H), lambda i: (0, 0)),
            pl.BlockSpec((H, H), lambda i: (0, 0)),
        ],
        out_specs=pl.BlockSpec((_BLK, H), lambda i: (i, 0)),
        out_shape=jax.ShapeDtypeStruct((Npad, H), jnp.float32),
    )(aggp, p_prev, degT, b, W)


def _final(aggp, p_prev, degT, b, batch_p, Wc, bc, N):
    Npad, H = p_prev.shape
    grid = Npad // _BLK

    def body(a_ref, p_ref, deg_ref, b_ref, bt_ref, wc_ref, bc_ref, y_ref,
             sums, cnts):
        i = pl.program_id(0)

        @pl.when(i == 0)
        def _():
            sums[...] = jnp.zeros_like(sums)
            cnts[...] = jnp.zeros_like(cnts)

        d = jnp.sum(deg_ref[...], axis=1, keepdims=True) + 1.0
        rows = lax.broadcasted_iota(jnp.int32, (_BLK, 1), 0) + i * _BLK
        valid = rows < N
        dinv = jnp.where(valid, lax.rsqrt(d), 0.0)
        h = jnp.maximum(dinv * (a_ref[0] + a_ref[1] - p_ref[...]) + b_ref[...],
                        0.0)
        M = jnp.where(
            (bt_ref[...] == lax.broadcasted_iota(jnp.int32, (_BLK, _G), 1))
            & valid, 1.0, 0.0)
        sums[...] += lax.dot_general(M, h, (((0,), (0,)), ((), ())),
                                     preferred_element_type=jnp.float32)
        cnts[...] += lax.dot_general(M, jnp.ones((_BLK, H), jnp.float32),
                                     (((0,), (0,)), ((), ())),
                                     preferred_element_type=jnp.float32)

        @pl.when(i == grid - 1)
        def _():
            pooled = sums[...] / jnp.maximum(cnts[...], 1.0)
            y_ref[...] = jnp.dot(pooled, wc_ref[...],
                                 preferred_element_type=jnp.float32) + bc_ref[...]

    return pl.pallas_call(
        body,
        grid=(grid,),
        in_specs=[
            pl.BlockSpec((_NC, _BLK, H), lambda i: (0, i, 0)),
            pl.BlockSpec((_BLK, H), lambda i: (i, 0)),
            pl.BlockSpec((_BLK, _NC), lambda i: (i, 0)),
            pl.BlockSpec((1, H), lambda i: (0, 0)),
            pl.BlockSpec((_BLK, 1), lambda i: (i, 0)),
            pl.BlockSpec((H, 1), lambda i: (0, 0)),
            pl.BlockSpec((1, 1), lambda i: (0, 0)),
        ],
        out_specs=pl.BlockSpec((_G, 1), lambda i: (0, 0)),
        out_shape=jax.ShapeDtypeStruct((_G, 1), jnp.float32),
        scratch_shapes=[
            pltpu.VMEM((_G, H), jnp.float32),
            pltpu.VMEM((_G, H), jnp.float32),
        ],
    )(aggp, p_prev, degT, b, batch_p, Wc, bc)


def kernel(x, edge_index, batch, W1, b1, W2, b2, Wc, bc):
    N, D = x.shape
    H = W1.shape[1]
    E = edge_index.shape[1]

    Npad = ((N + 1 + _BLK - 1) // _BLK) * _BLK
    EW = ((E // _NW + _B - 1) // _B) * _B
    Epad = EW * _NW

    src = jnp.concatenate(
        [edge_index[0], jnp.full((Epad - E,), N, jnp.int32)])
    dst = jnp.concatenate(
        [edge_index[1], jnp.full((Epad - E,), N, jnp.int32)])
    x_p = jnp.zeros((Npad, D), jnp.float32).at[:N].set(x)
    batch_p = jnp.full((Npad, 1), _G, jnp.int32).at[:N, 0].set(batch)

    deg_parts = _make_deg(Npad, Epad)(dst)
    degT = deg_parts.T

    agg = _make_agg(Npad, D, Epad)

    p1 = _mm_scale(x_p, W1, degT, N)
    a1 = agg(p1, src, dst)
    p2 = _layer_mm(a1, p1, degT, b1.reshape(1, H), W2, N)
    a2 = agg(p2, src, dst)
    y = _final(a2, p2, degT, b2.reshape(1, H), batch_p,
               Wc, bc.reshape(1, 1), N)
    return y
